# initial kernel scaffold (unmeasured)
import jax
import jax.numpy as jnp
from jax import lax
from jax.experimental import pallas as pl
from jax.experimental.pallas import tpu as pltpu

N_DEV = 8


def kernel(x, Wq, K_ext, V_ext, Wo):
    B_loc, Sq, D = x.shape
    _, HD_loc = Wq.shape
    B_glob, Skv, Hq, Dh = K_ext.shape
    H_loc = HD_loc // Dh

    def body(x_ref, wq_ref, k_hbm, v_hbm, wo_ref, out_ref,
             wq_buf, wo_buf, k_chunk, v_chunk,
             wq_send, wq_recv, wo_send, wo_recv, kv_sems):
        my = lax.axis_index("i")
        right = lax.rem(my + 1, N_DEV)
        left = lax.rem(my + N_DEV - 1, N_DEV)

        barrier = pltpu.get_barrier_semaphore()
        for nbr in (left, right):
            pl.semaphore_signal(
                barrier, inc=1,
                device_id=(nbr,), device_id_type=pl.DeviceIdType.MESH,
            )
        pl.semaphore_wait(barrier, 2)

        wq_buf[0] = wq_ref[...]
        wo_buf[0] = wo_ref[...]

        out_ref[...] = jnp.zeros((B_loc, Sq, D), jnp.float32)
        x2 = x_ref[...].reshape(B_loc * Sq, D)
        b0 = my * B_loc

        def compute_chunk(slot, g):
            ck = pltpu.make_async_copy(
                k_hbm.at[pl.ds(b0, B_loc), :, pl.ds(g * H_loc, H_loc), :],
                k_chunk, kv_sems.at[0])
            cv = pltpu.make_async_copy(
                v_hbm.at[pl.ds(b0, B_loc), :, pl.ds(g * H_loc, H_loc), :],
                v_chunk, kv_sems.at[1])
            ck.start()
            cv.start()
            ck.wait()
            cv.wait()

            wq_g = wq_buf[slot]
            wo_g = wo_buf[slot]
            q2 = jnp.dot(x2, wq_g, preferred_element_type=jnp.float32)
            for b in range(B_loc):
                for h in range(H_loc):
                    q = q2[b * Sq:(b + 1) * Sq, h * Dh:(h + 1) * Dh]
                    k = k_chunk[b, :, h, :]
                    v = v_chunk[b, :, h, :]
                    s = lax.dot_general(
                        q, k, (((1,), (1,)), ((), ())),
                        preferred_element_type=jnp.float32) * 0.125
                    mx = jnp.max(s, axis=-1, keepdims=True)
                    w = jnp.exp(s - mx)
                    w = w / jnp.sum(w, axis=-1, keepdims=True)
                    ctx = jnp.dot(w, v, preferred_element_type=jnp.float32)
                    contrib = jnp.dot(ctx, wo_g[h * Dh:(h + 1) * Dh, :],
                                      preferred_element_type=jnp.float32)
                    out_ref[b] = out_ref[b] + contrib

        for h in range(N_DEV - 1):
            rq = pltpu.make_async_remote_copy(
                src_ref=wq_buf.at[h], dst_ref=wq_buf.at[h + 1],
                send_sem=wq_send.at[h], recv_sem=wq_recv.at[h + 1],
                device_id=(right,), device_id_type=pl.DeviceIdType.MESH)
            ro = pltpu.make_async_remote_copy(
                src_ref=wo_buf.at[h], dst_ref=wo_buf.at[h + 1],
                send_sem=wo_send.at[h], recv_sem=wo_recv.at[h + 1],
                device_id=(right,), device_id_type=pl.DeviceIdType.MESH)
            rq.start()
            ro.start()
            compute_chunk(h, lax.rem(my + (N_DEV - h), N_DEV))
            rq.wait()
            ro.wait()
        compute_chunk(N_DEV - 1, lax.rem(my + 1, N_DEV))

    return pl.pallas_call(
        body,
        out_shape=jax.ShapeDtypeStruct((B_loc, Sq, D), jnp.float32),
        in_specs=[
            pl.BlockSpec(memory_space=pltpu.VMEM),
            pl.BlockSpec(memory_space=pltpu.VMEM),
            pl.BlockSpec(memory_space=pltpu.ANY),
            pl.BlockSpec(memory_space=pltpu.ANY),
            pl.BlockSpec(memory_space=pltpu.VMEM),
        ],
        out_specs=pl.BlockSpec(memory_space=pltpu.VMEM),
        scratch_shapes=[
            pltpu.VMEM((N_DEV, D, HD_loc), jnp.float32),
            pltpu.VMEM((N_DEV, HD_loc, D), jnp.float32),
            pltpu.VMEM((B_loc, Skv, H_loc, Dh), jnp.float32),
            pltpu.VMEM((B_loc, Skv, H_loc, Dh), jnp.float32),
            pltpu.SemaphoreType.DMA((N_DEV,)),
            pltpu.SemaphoreType.DMA((N_DEV,)),
            pltpu.SemaphoreType.DMA((N_DEV,)),
            pltpu.SemaphoreType.DMA((N_DEV,)),
            pltpu.SemaphoreType.DMA((2,)),
        ],
        compiler_params=pltpu.CompilerParams(collective_id=0),
    )(x, Wq, K_ext, V_ext, Wo)


# baseline (device time: 158330 ns/iter reference)
import jax
import jax.numpy as jnp
from jax import lax
from jax.experimental import pallas as pl
from jax.experimental.pallas import tpu as pltpu

N_DEV = 8


def kernel(x, Wq, K_ext, V_ext, Wo):
    B_loc, Sq, D = x.shape
    _, HD_loc = Wq.shape
    B_glob, Skv, Hq, Dh = K_ext.shape
    H_loc = HD_loc // Dh

    def body(x_ref, wq_ref, k_hbm, v_hbm, wo_ref, out_ref,
             wq_buf, wo_buf, k_chunk, v_chunk,
             wq_send, wq_recv, wo_send, wo_recv, kv_sems):
        my = lax.axis_index("i")
        right = lax.rem(my + 1, N_DEV)
        left = lax.rem(my + N_DEV - 1, N_DEV)

        barrier = pltpu.get_barrier_semaphore()
        for nbr in (left, right):
            pl.semaphore_signal(
                barrier, inc=1,
                device_id=(nbr,), device_id_type=pl.DeviceIdType.MESH,
            )
        pl.semaphore_wait(barrier, 2)

        wq_buf[0] = wq_ref[...]
        wo_buf[0] = wo_ref[...]

        out_ref[...] = jnp.zeros((B_loc, Sq, D), jnp.float32)
        x2 = x_ref[...].reshape(B_loc * Sq, D)
        b0 = my * B_loc

        def compute_chunk(slot, g):
            ck = pltpu.make_async_copy(
                k_hbm.at[pl.ds(b0, B_loc), :, pl.ds(g * H_loc, H_loc), :],
                k_chunk, kv_sems.at[0])
            cv = pltpu.make_async_copy(
                v_hbm.at[pl.ds(b0, B_loc), :, pl.ds(g * H_loc, H_loc), :],
                v_chunk, kv_sems.at[1])
            ck.start()
            cv.start()
            ck.wait()
            cv.wait()

            wq_g = wq_buf[slot]
            wo_g = wo_buf[slot]
            q2 = jnp.dot(x2, wq_g, preferred_element_type=jnp.float32)
            for b in range(B_loc):
                for h in range(H_loc):
                    q = q2[b * Sq:(b + 1) * Sq, h * Dh:(h + 1) * Dh]
                    k = k_chunk[b, :, h, :]
                    v = v_chunk[b, :, h, :]
                    s = lax.dot_general(
                        q, k, (((1,), (1,)), ((), ())),
                        preferred_element_type=jnp.float32) * 0.125
                    mx = jnp.max(s, axis=-1, keepdims=True)
                    w = jnp.exp(s - mx)
                    w = w / jnp.sum(w, axis=-1, keepdims=True)
                    ctx = jnp.dot(w, v, preferred_element_type=jnp.float32)
                    contrib = jnp.dot(ctx, wo_g[h * Dh:(h + 1) * Dh, :],
                                      preferred_element_type=jnp.float32)
                    out_ref[b] = out_ref[b] + contrib

        for h in range(N_DEV - 1):
            rq = pltpu.make_async_remote_copy(
                src_ref=wq_buf.at[h], dst_ref=wq_buf.at[h + 1],
                send_sem=wq_send.at[h], recv_sem=wq_recv.at[h + 1],
                device_id=(right,), device_id_type=pl.DeviceIdType.MESH)
            ro = pltpu.make_async_remote_copy(
                src_ref=wo_buf.at[h], dst_ref=wo_buf.at[h + 1],
                send_sem=wo_send.at[h], recv_sem=wo_recv.at[h + 1],
                device_id=(right,), device_id_type=pl.DeviceIdType.MESH)
            rq.start()
            ro.start()
            compute_chunk(h, lax.rem(my + (N_DEV - h), N_DEV))
            rq.wait()
            ro.wait()
        compute_chunk(N_DEV - 1, lax.rem(my + 1, N_DEV))

    return pl.pallas_call(
        body,
        out_shape=jax.ShapeDtypeStruct((B_loc, Sq, D), jnp.float32),
        in_specs=[
            pl.BlockSpec(memory_space=pltpu.VMEM),
            pl.BlockSpec(memory_space=pltpu.VMEM),
            pl.BlockSpec(memory_space=pl.ANY),
            pl.BlockSpec(memory_space=pl.ANY),
            pl.BlockSpec(memory_space=pltpu.VMEM),
        ],
        out_specs=pl.BlockSpec(memory_space=pltpu.VMEM),
        scratch_shapes=[
            pltpu.VMEM((N_DEV, D, HD_loc), jnp.float32),
            pltpu.VMEM((N_DEV, HD_loc, D), jnp.float32),
            pltpu.VMEM((B_loc, Skv, H_loc, Dh), jnp.float32),
            pltpu.VMEM((B_loc, Skv, H_loc, Dh), jnp.float32),
            pltpu.SemaphoreType.DMA((N_DEV,)),
            pltpu.SemaphoreType.DMA((N_DEV,)),
            pltpu.SemaphoreType.DMA((N_DEV,)),
            pltpu.SemaphoreType.DMA((N_DEV,)),
            pltpu.SemaphoreType.DMA((2,)),
        ],
        compiler_params=pltpu.CompilerParams(collective_id=0),
    )(x, Wq, K_ext, V_ext, Wo)


# device time: 97404 ns/iter; 1.6255x vs baseline; 1.6255x over previous
import jax
import jax.numpy as jnp
from jax import lax
from jax.experimental import pallas as pl
from jax.experimental.pallas import tpu as pltpu

N_DEV = 8
R_HOPS = 4
L_HOPS = 3

_SCHEDULE = {
    0: [("r", 0, 0, 0)],
    1: [("r", 1, -1, 1), ("l", 1, 1, 2)],
    2: [("r", 2, -2, 3), ("l", 2, 2, 4)],
    3: [("r", 3, -3, 5), ("l", 3, 3, 6)],
    4: [("r", 4, -4, 7)],
}
_OFFSETS = [0, -1, 1, -2, 2, -3, 3, -4]


def kernel(x, Wq, K_ext, V_ext, Wo):
    B_loc, Sq, D = x.shape
    _, HD_loc = Wq.shape
    B_glob, Skv, Hq, Dh = K_ext.shape
    H_loc = HD_loc // Dh
    BS = B_loc * Sq

    def body(x_ref, wq_ref, k_hbm, v_hbm, wo_ref, out_ref,
             rwq, rwo, lwq, lwo, kc, vc,
             rwq_s, rwq_r, rwo_s, rwo_r,
             lwq_s, lwq_r, lwo_s, lwo_r, ks, vs):
        my = lax.axis_index("i")
        right = lax.rem(my + 1, N_DEV)
        left = lax.rem(my + N_DEV - 1, N_DEV)
        b0 = my * B_loc

        kv_copies = []
        for j, off in enumerate(_OFFSETS):
            g = lax.rem(my + off + N_DEV, N_DEV)
            ck = pltpu.make_async_copy(
                k_hbm.at[pl.ds(b0, B_loc), :, pl.ds(g * H_loc, H_loc), :],
                kc.at[j], ks.at[j])
            cv = pltpu.make_async_copy(
                v_hbm.at[pl.ds(b0, B_loc), :, pl.ds(g * H_loc, H_loc), :],
                vc.at[j], vs.at[j])
            ck.start()
            cv.start()
            kv_copies.append((ck, cv))

        barrier = pltpu.get_barrier_semaphore()
        for nbr in (left, right):
            pl.semaphore_signal(
                barrier, inc=1,
                device_id=(nbr,), device_id_type=pl.DeviceIdType.MESH,
            )
        pl.semaphore_wait(barrier, 2)

        wq_own = wq_ref[...].astype(jnp.bfloat16)
        wo_own = wo_ref[...].astype(jnp.bfloat16)
        rwq[0] = wq_own
        rwo[0] = wo_own
        lwq[0] = wq_own
        lwo[0] = wo_own

        x2 = x_ref[...].reshape(BS, D)

        def compute_chunk(buf_wq, buf_wo, slot, j):
            wq_g = buf_wq[slot].astype(jnp.float32)
            wo_g = buf_wo[slot].astype(jnp.float32)
            q2 = jnp.dot(x2, wq_g, preferred_element_type=jnp.float32) * 0.125
            ck, cv = kv_copies[j]
            ck.wait()
            cv.wait()
            ctx_rows = []
            for b in range(B_loc):
                ctx_heads = []
                for h in range(H_loc):
                    q = q2[b * Sq:(b + 1) * Sq, h * Dh:(h + 1) * Dh]
                    k = kc[j, b, :, h, :]
                    v = vc[j, b, :, h, :]
                    s = lax.dot_general(
                        q, k, (((1,), (1,)), ((), ())),
                        preferred_element_type=jnp.float32)
                    mx = jnp.max(s, axis=-1, keepdims=True)
                    w = jnp.exp(s - mx)
                    w = w / jnp.sum(w, axis=-1, keepdims=True)
                    ctx_heads.append(
                        jnp.dot(w, v, preferred_element_type=jnp.float32))
                ctx_rows.append(jnp.concatenate(ctx_heads, axis=1))
            ctx2 = jnp.concatenate(ctx_rows, axis=0)
            return jnp.dot(ctx2, wo_g, preferred_element_type=jnp.float32)

        acc = jnp.zeros((BS, D), jnp.float32)
        for hop in range(R_HOPS + 1):
            started = []
            if hop < R_HOPS:
                for buf, s_sem, r_sem in ((rwq, rwq_s, rwq_r),
                                          (rwo, rwo_s, rwo_r)):
                    d = pltpu.make_async_remote_copy(
                        src_ref=buf.at[hop], dst_ref=buf.at[hop + 1],
                        send_sem=s_sem.at[hop], recv_sem=r_sem.at[hop],
                        device_id=(right,),
                        device_id_type=pl.DeviceIdType.MESH)
                    d.start()
                    started.append(d)
            if hop < L_HOPS:
                for buf, s_sem, r_sem in ((lwq, lwq_s, lwq_r),
                                          (lwo, lwo_s, lwo_r)):
                    d = pltpu.make_async_remote_copy(
                        src_ref=buf.at[hop], dst_ref=buf.at[hop + 1],
                        send_sem=s_sem.at[hop], recv_sem=r_sem.at[hop],
                        device_id=(left,),
                        device_id_type=pl.DeviceIdType.MESH)
                    d.start()
                    started.append(d)
            for kind, slot, _off, j in _SCHEDULE[hop]:
                if kind == "r":
                    acc = acc + compute_chunk(rwq, rwo, slot, j)
                else:
                    acc = acc + compute_chunk(lwq, lwo, slot, j)
            for d in started:
                d.wait()

        out_ref[...] = acc.reshape(B_loc, Sq, D)

    return pl.pallas_call(
        body,
        out_shape=jax.ShapeDtypeStruct((B_loc, Sq, D), jnp.float32),
        in_specs=[
            pl.BlockSpec(memory_space=pltpu.VMEM),
            pl.BlockSpec(memory_space=pltpu.VMEM),
            pl.BlockSpec(memory_space=pl.ANY),
            pl.BlockSpec(memory_space=pl.ANY),
            pl.BlockSpec(memory_space=pltpu.VMEM),
        ],
        out_specs=pl.BlockSpec(memory_space=pltpu.VMEM),
        scratch_shapes=[
            pltpu.VMEM((R_HOPS + 1, D, HD_loc), jnp.bfloat16),
            pltpu.VMEM((R_HOPS + 1, HD_loc, D), jnp.bfloat16),
            pltpu.VMEM((L_HOPS + 1, D, HD_loc), jnp.bfloat16),
            pltpu.VMEM((L_HOPS + 1, HD_loc, D), jnp.bfloat16),
            pltpu.VMEM((N_DEV, B_loc, Skv, H_loc, Dh), jnp.float32),
            pltpu.VMEM((N_DEV, B_loc, Skv, H_loc, Dh), jnp.float32),
            pltpu.SemaphoreType.DMA((R_HOPS,)),
            pltpu.SemaphoreType.DMA((R_HOPS,)),
            pltpu.SemaphoreType.DMA((R_HOPS,)),
            pltpu.SemaphoreType.DMA((R_HOPS,)),
            pltpu.SemaphoreType.DMA((L_HOPS,)),
            pltpu.SemaphoreType.DMA((L_HOPS,)),
            pltpu.SemaphoreType.DMA((L_HOPS,)),
            pltpu.SemaphoreType.DMA((L_HOPS,)),
            pltpu.SemaphoreType.DMA((N_DEV,)),
            pltpu.SemaphoreType.DMA((N_DEV,)),
        ],
        compiler_params=pltpu.CompilerParams(collective_id=0),
    )(x, Wq, K_ext, V_ext, Wo)


# device time: 96919 ns/iter; 1.6336x vs baseline; 1.0050x over previous
import jax
import jax.numpy as jnp
from jax import lax
from jax.experimental import pallas as pl
from jax.experimental.pallas import tpu as pltpu

N_DEV = 8
R_HOPS = 4
L_HOPS = 3

_SCHEDULE = {
    0: [("r", 0, 0, 0)],
    1: [("r", 1, -1, 1), ("l", 1, 1, 2)],
    2: [("r", 2, -2, 3), ("l", 2, 2, 4)],
    3: [("r", 3, -3, 5), ("l", 3, 3, 6)],
    4: [("r", 4, -4, 7)],
}
_OFFSETS = [0, -1, 1, -2, 2, -3, 3, -4]


def kernel(x, Wq, K_ext, V_ext, Wo):
    B_loc, Sq, D = x.shape
    _, HD_loc = Wq.shape
    B_glob, Skv, Hq, Dh = K_ext.shape
    H_loc = HD_loc // Dh
    BS = B_loc * Sq

    def body(x_ref, wq_ref, k_hbm, v_hbm, wo_ref, out_ref,
             rwq, rwo, lwq, lwo, kc, vc,
             rwq_s, rwq_r, rwo_s, rwo_r,
             lwq_s, lwq_r, lwo_s, lwo_r, ks, vs):
        my = lax.axis_index("i")
        right = lax.rem(my + 1, N_DEV)
        left = lax.rem(my + N_DEV - 1, N_DEV)
        b0 = my * B_loc

        kv_copies = []
        for j, off in enumerate(_OFFSETS):
            g = lax.rem(my + off + N_DEV, N_DEV)
            ck = pltpu.make_async_copy(
                k_hbm.at[pl.ds(b0, B_loc), :, pl.ds(g * H_loc, H_loc), :],
                kc.at[j], ks.at[j])
            cv = pltpu.make_async_copy(
                v_hbm.at[pl.ds(b0, B_loc), :, pl.ds(g * H_loc, H_loc), :],
                vc.at[j], vs.at[j])
            ck.start()
            cv.start()
            kv_copies.append((ck, cv))

        barrier = pltpu.get_barrier_semaphore()
        for nbr in (left, right):
            pl.semaphore_signal(
                barrier, inc=1,
                device_id=(nbr,), device_id_type=pl.DeviceIdType.MESH,
            )
        pl.semaphore_wait(barrier, 2)

        wq_own = wq_ref[...].astype(jnp.bfloat16)
        wo_own = wo_ref[...].astype(jnp.bfloat16)
        rwq[0] = wq_own
        rwo[0] = wo_own
        lwq[0] = wq_own
        lwo[0] = wo_own

        x2 = x_ref[...].reshape(BS, D).astype(jnp.bfloat16)

        rows = lax.broadcasted_iota(jnp.int32, (BS, BS), 0) // Sq
        cols = lax.broadcasted_iota(jnp.int32, (BS, BS), 1) // Sq
        bmask = (rows == cols).astype(jnp.float32)

        def compute_chunk(buf_wq, buf_wo, slot, j):
            wq_g = buf_wq[slot]
            wo_g = buf_wo[slot]
            q2 = (jnp.dot(x2, wq_g, preferred_element_type=jnp.float32)
                  * 0.125).astype(jnp.bfloat16)
            ck, cv = kv_copies[j]
            ck.wait()
            cv.wait()
            ctx_heads = []
            for h in range(H_loc):
                q = q2[:, h * Dh:(h + 1) * Dh]
                k = kc[j, :, :, h, :].reshape(BS, Dh).astype(jnp.bfloat16)
                v = vc[j, :, :, h, :].reshape(BS, Dh).astype(jnp.bfloat16)
                s = lax.dot_general(
                    q, k, (((1,), (1,)), ((), ())),
                    preferred_element_type=jnp.float32)
                w = jnp.exp(s) * bmask
                wsum = jnp.sum(w, axis=-1, keepdims=True)
                ctx = jnp.dot(w.astype(jnp.bfloat16), v,
                              preferred_element_type=jnp.float32)
                ctx_heads.append(ctx / wsum)
            ctx2 = jnp.concatenate(ctx_heads, axis=1).astype(jnp.bfloat16)
            return jnp.dot(ctx2, wo_g, preferred_element_type=jnp.float32)

        acc = jnp.zeros((BS, D), jnp.float32)
        for hop in range(R_HOPS + 1):
            started = []
            if hop < R_HOPS:
                for buf, s_sem, r_sem in ((rwq, rwq_s, rwq_r),
                                          (rwo, rwo_s, rwo_r)):
                    d = pltpu.make_async_remote_copy(
                        src_ref=buf.at[hop], dst_ref=buf.at[hop + 1],
                        send_sem=s_sem.at[hop], recv_sem=r_sem.at[hop],
                        device_id=(right,),
                        device_id_type=pl.DeviceIdType.MESH)
                    d.start()
                    started.append(d)
            if hop < L_HOPS:
                for buf, s_sem, r_sem in ((lwq, lwq_s, lwq_r),
                                          (lwo, lwo_s, lwo_r)):
                    d = pltpu.make_async_remote_copy(
                        src_ref=buf.at[hop], dst_ref=buf.at[hop + 1],
                        send_sem=s_sem.at[hop], recv_sem=r_sem.at[hop],
                        device_id=(left,),
                        device_id_type=pl.DeviceIdType.MESH)
                    d.start()
                    started.append(d)
            for kind, slot, _off, j in _SCHEDULE[hop]:
                if kind == "r":
                    acc = acc + compute_chunk(rwq, rwo, slot, j)
                else:
                    acc = acc + compute_chunk(lwq, lwo, slot, j)
            for d in started:
                d.wait()

        out_ref[...] = acc.reshape(B_loc, Sq, D)

    return pl.pallas_call(
        body,
        out_shape=jax.ShapeDtypeStruct((B_loc, Sq, D), jnp.float32),
        in_specs=[
            pl.BlockSpec(memory_space=pltpu.VMEM),
            pl.BlockSpec(memory_space=pltpu.VMEM),
            pl.BlockSpec(memory_space=pl.ANY),
            pl.BlockSpec(memory_space=pl.ANY),
            pl.BlockSpec(memory_space=pltpu.VMEM),
        ],
        out_specs=pl.BlockSpec(memory_space=pltpu.VMEM),
        scratch_shapes=[
            pltpu.VMEM((R_HOPS + 1, D, HD_loc), jnp.bfloat16),
            pltpu.VMEM((R_HOPS + 1, HD_loc, D), jnp.bfloat16),
            pltpu.VMEM((L_HOPS + 1, D, HD_loc), jnp.bfloat16),
            pltpu.VMEM((L_HOPS + 1, HD_loc, D), jnp.bfloat16),
            pltpu.VMEM((N_DEV, B_loc, Skv, H_loc, Dh), jnp.float32),
            pltpu.VMEM((N_DEV, B_loc, Skv, H_loc, Dh), jnp.float32),
            pltpu.SemaphoreType.DMA((R_HOPS,)),
            pltpu.SemaphoreType.DMA((R_HOPS,)),
            pltpu.SemaphoreType.DMA((R_HOPS,)),
            pltpu.SemaphoreType.DMA((R_HOPS,)),
            pltpu.SemaphoreType.DMA((L_HOPS,)),
            pltpu.SemaphoreType.DMA((L_HOPS,)),
            pltpu.SemaphoreType.DMA((L_HOPS,)),
            pltpu.SemaphoreType.DMA((L_HOPS,)),
            pltpu.SemaphoreType.DMA((N_DEV,)),
            pltpu.SemaphoreType.DMA((N_DEV,)),
        ],
        compiler_params=pltpu.CompilerParams(collective_id=0),
    )(x, Wq, K_ext, V_ext, Wo)


# device time: 42149 ns/iter; 3.7564x vs baseline; 2.2994x over previous
import jax
import jax.numpy as jnp
from jax import lax
from jax.experimental import pallas as pl
from jax.experimental.pallas import tpu as pltpu

N_DEV = 8
R_HOPS = 3
L_HOPS = 3

def kernel(x, Wq, K_ext, V_ext, Wo):
    B_loc, Sq, D = x.shape
    _, HD_loc = Wq.shape
    B_glob, Skv, Hq, Dh = K_ext.shape
    H_loc = HD_loc // Dh
    BS = B_loc * Sq

    def body(x_ref, wq_ref, k_ref, v_ref, wo_ref, out_ref,
             rwq, rwo, lwq, lwo, zwq, zwo, kg, vg,
             rwq_s, rwq_r, rwo_s, rwo_r,
             lwq_s, lwq_r, lwo_s, lwo_r,
             zwq_s, zwq_r, zwo_s, zwo_r):
        my = lax.axis_index("i")
        right = lax.rem(my + 1, N_DEV)
        left = lax.rem(my + N_DEV - 1, N_DEV)
        anti = lax.rem(my + N_DEV // 2, N_DEV)

        barrier = pltpu.get_barrier_semaphore()
        for nbr in (left, right, anti):
            pl.semaphore_signal(
                barrier, inc=1,
                device_id=(nbr,), device_id_type=pl.DeviceIdType.MESH,
            )
        pl.semaphore_wait(barrier, 3)

        wq_own = wq_ref[...].astype(jnp.bfloat16)
        wo_own = wo_ref[...].astype(jnp.bfloat16)
        rwq[0] = wq_own
        rwo[0] = wo_own
        lwq[0] = wq_own
        lwo[0] = wo_own

        x2 = x_ref[...].reshape(BS, D).astype(jnp.bfloat16)

        rows = lax.broadcasted_iota(jnp.int32, (BS, BS), 0) // Sq
        cols = lax.broadcasted_iota(jnp.int32, (BS, BS), 1) // Sq
        bmask = (rows == cols).astype(jnp.float32)

        def compute_chunk(buf_wq, buf_wo, slot, g):
            wq_g = buf_wq[slot]
            wo_g = buf_wo[slot]
            q2 = (jnp.dot(x2, wq_g, preferred_element_type=jnp.float32)
                  * 0.125).astype(jnp.bfloat16)
            kj = kg[g]
            vj = vg[g]
            ctx_heads = []
            for h in range(H_loc):
                q = q2[:, h * Dh:(h + 1) * Dh]
                k = kj[:, h * Dh:(h + 1) * Dh]
                v = vj[:, h * Dh:(h + 1) * Dh]
                s = lax.dot_general(
                    q, k, (((1,), (1,)), ((), ())),
                    preferred_element_type=jnp.float32)
                w = jnp.exp(s) * bmask
                wsum = jnp.sum(w, axis=-1, keepdims=True)
                ctx = jnp.dot(w.astype(jnp.bfloat16), v,
                              preferred_element_type=jnp.float32)
                ctx_heads.append(ctx / wsum)
            ctx2 = jnp.concatenate(ctx_heads, axis=1).astype(jnp.bfloat16)
            return jnp.dot(ctx2, wo_g, preferred_element_type=jnp.float32)

        chains = (
            (rwq, rwq_s, rwq_r, right, R_HOPS),
            (lwq, lwq_s, lwq_r, left, L_HOPS),
            (rwo, rwo_s, rwo_r, right, R_HOPS),
            (lwo, lwo_s, lwo_r, left, L_HOPS),
        )

        def chain_send(ci, hop):
            buf, s_sem, r_sem, dev, _ = chains[ci]
            d = pltpu.make_async_remote_copy(
                src_ref=buf.at[hop], dst_ref=buf.at[hop + 1],
                send_sem=s_sem.at[hop], recv_sem=r_sem.at[hop],
                device_id=(dev,), device_id_type=pl.DeviceIdType.MESH)
            d.start()
            return d

        acc = jnp.zeros((BS, D), jnp.float32)
        done = []
        prev = [chain_send(ci, 0) for ci in range(len(chains))]

        z_descs = []
        for src, dst, s_sem, r_sem in (
                (rwq.at[0], zwq.at[0], zwq_s.at[0], zwq_r.at[0]),
                (rwo.at[0], zwo.at[0], zwo_s.at[0], zwo_r.at[0])):
            d = pltpu.make_async_remote_copy(
                src_ref=src, dst_ref=dst,
                send_sem=s_sem, recv_sem=r_sem,
                device_id=(anti,), device_id_type=pl.DeviceIdType.MESH)
            d.start()
            z_descs.append(d)

        kvv = k_ref[...]
        vvv = v_ref[...]
        for G in range(N_DEV):
            kg[G] = kvv[:, G * HD_loc:(G + 1) * HD_loc]
            vg[G] = vvv[:, G * HD_loc:(G + 1) * HD_loc]
        acc = acc + compute_chunk(rwq, rwo, 0, my)

        for hop in range(1, R_HOPS + 1):
            for ci, chain in enumerate(chains):
                prev[ci].wait_recv()
                done.append(prev[ci])
                if hop < chain[4]:
                    prev[ci] = chain_send(ci, hop)
            acc = acc + compute_chunk(
                rwq, rwo, hop, lax.rem(my + N_DEV - hop, N_DEV))
            acc = acc + compute_chunk(
                lwq, lwo, hop, lax.rem(my + hop, N_DEV))
        for d in z_descs:
            d.wait_recv()
        acc = acc + compute_chunk(
            zwq, zwo, 0, lax.rem(my + N_DEV // 2, N_DEV))

        out_ref[...] = acc.reshape(B_loc, Sq, D)
        for d in done + z_descs:
            d.wait_send()

    b0 = lax.axis_index("i") * B_loc
    k_loc = lax.dynamic_slice_in_dim(K_ext, b0, B_loc, axis=0)
    v_loc = lax.dynamic_slice_in_dim(V_ext, b0, B_loc, axis=0)
    k_loc = k_loc.reshape(BS, Hq * Dh).astype(jnp.bfloat16)
    v_loc = v_loc.reshape(BS, Hq * Dh).astype(jnp.bfloat16)

    return pl.pallas_call(
        body,
        out_shape=jax.ShapeDtypeStruct((B_loc, Sq, D), jnp.float32),
        in_specs=[
            pl.BlockSpec(memory_space=pltpu.VMEM),
            pl.BlockSpec(memory_space=pltpu.VMEM),
            pl.BlockSpec(memory_space=pltpu.VMEM),
            pl.BlockSpec(memory_space=pltpu.VMEM),
            pl.BlockSpec(memory_space=pltpu.VMEM),
        ],
        out_specs=pl.BlockSpec(memory_space=pltpu.VMEM),
        scratch_shapes=[
            pltpu.VMEM((R_HOPS + 1, D, HD_loc), jnp.bfloat16),
            pltpu.VMEM((R_HOPS + 1, HD_loc, D), jnp.bfloat16),
            pltpu.VMEM((L_HOPS + 1, D, HD_loc), jnp.bfloat16),
            pltpu.VMEM((L_HOPS + 1, HD_loc, D), jnp.bfloat16),
            pltpu.VMEM((1, D, HD_loc), jnp.bfloat16),
            pltpu.VMEM((1, HD_loc, D), jnp.bfloat16),
            pltpu.VMEM((N_DEV, BS, HD_loc), jnp.bfloat16),
            pltpu.VMEM((N_DEV, BS, HD_loc), jnp.bfloat16),
            pltpu.SemaphoreType.DMA((R_HOPS,)),
            pltpu.SemaphoreType.DMA((R_HOPS,)),
            pltpu.SemaphoreType.DMA((R_HOPS,)),
            pltpu.SemaphoreType.DMA((R_HOPS,)),
            pltpu.SemaphoreType.DMA((L_HOPS,)),
            pltpu.SemaphoreType.DMA((L_HOPS,)),
            pltpu.SemaphoreType.DMA((L_HOPS,)),
            pltpu.SemaphoreType.DMA((L_HOPS,)),
            pltpu.SemaphoreType.DMA((1,)),
            pltpu.SemaphoreType.DMA((1,)),
            pltpu.SemaphoreType.DMA((1,)),
            pltpu.SemaphoreType.DMA((1,)),
        ],
        compiler_params=pltpu.CompilerParams(collective_id=0),
    )(x, Wq, k_loc, v_loc, Wo)
